# baseline (speedup 1.0000x reference)
import functools

import jax
import jax.numpy as jnp
from jax import lax
from jax.experimental import pallas as pl
from jax.experimental.pallas import tpu as pltpu
from jax.experimental.pallas import tpu_sc as plsc

B = 2
H = 512
W = 512
HW = H * W
FEAT_DIM = 96
EMBED_DIM = 64
MAX_SEG = 32

PIX_BLK = 32768

N_WORKERS = 32
CHUNK = B * HW // N_WORKERS
WORKERS_PER_B = N_WORKERS // B


def _sc_hist_body(lab_hbm, out_hbm, lab_v, cnt_v):
    wid = lax.axis_index("s") * 2 + lax.axis_index("c")
    base = wid * CHUNK
    pltpu.sync_copy(lab_hbm.at[pl.ds(base, CHUNK)], lab_v)
    cnt_v[pl.ds(0, 16)] = jnp.zeros((16,), jnp.float32)
    cnt_v[pl.ds(16, 16)] = jnp.zeros((16,), jnp.float32)
    ones = jnp.ones((16,), jnp.float32)

    @pl.loop(0, CHUNK // 16)
    def _count(i):
        lab = lab_v[pl.ds(i * 16, 16)]
        plsc.addupdate_scatter(cnt_v, [lab], ones)

    pltpu.sync_copy(cnt_v, out_hbm.at[wid])


def _sc_histogram(labels_flat):
    mesh = plsc.VectorSubcoreMesh(core_axis_name="c", subcore_axis_name="s")
    k = functools.partial(
        pl.kernel,
        mesh=mesh,
        out_type=jax.ShapeDtypeStruct((N_WORKERS, MAX_SEG), jnp.float32),
        scratch_types=[
            pltpu.VMEM((CHUNK,), jnp.int32),
            pltpu.VMEM((MAX_SEG,), jnp.float32),
        ],
        compiler_params=pltpu.CompilerParams(needs_layout_passes=False),
    )(_sc_hist_body)
    return k(labels_flat)


def _main_body(f_ref, l_ref, w1_ref, b1_ref, sums_ref):
    t = pl.program_id(1)

    @pl.when(t == 0)
    def _init():
        sums_ref[...] = jnp.zeros_like(sums_ref)

    x = f_ref[0].astype(jnp.bfloat16)
    w1 = w1_ref[...].astype(jnp.bfloat16)
    b1 = b1_ref[...]
    h = jax.lax.dot_general(w1, x, (((1,), (0,)), ((), ())),
                            preferred_element_type=jnp.float32)
    h = jnp.maximum(h + b1, 0.0).astype(jnp.bfloat16)

    lab = l_ref[0]
    sid = jax.lax.broadcasted_iota(jnp.int32, (MAX_SEG, PIX_BLK), 0)
    oh = (lab == sid).astype(jnp.bfloat16)

    sums_ref[0] += jax.lax.dot_general(h, oh, (((1,), (1,)), ((), ())),
                                       preferred_element_type=jnp.float32)


def _epilogue_body(sums_ref, hist_ref, table_ref, w2_ref, b2_ref, wout_ref,
                   bout_ref, out_ref):
    w2 = w2_ref[...]
    wout = wout_ref[...]
    wa = wout[:, :EMBED_DIM]
    wb = wout[:, EMBED_DIM:]
    b2 = b2_ref[...]
    bout = bout_ref[...]
    emb = table_ref[...][:MAX_SEG]

    hp = jnp.float32
    hi = jax.lax.Precision.HIGHEST
    g = jax.lax.dot_general(w2, wa, (((0,), (1,)), ((), ())),
                            preferred_element_type=hp, precision=hi)
    const = jax.lax.dot_general(wb, emb, (((1,), (1,)), ((), ())),
                                preferred_element_type=hp, precision=hi)
    const = const + jax.lax.dot_general(
        wa, b2, (((1,), (0,)), ((), ())), preferred_element_type=hp,
        precision=hi) + bout

    jj = jax.lax.broadcasted_iota(jnp.int32, (MAX_SEG, MAX_SEG), 0)
    ii = jax.lax.broadcasted_iota(jnp.int32, (MAX_SEG, MAX_SEG), 1)
    tri = (jj <= ii).astype(jnp.float32)
    dd = jax.lax.broadcasted_iota(jnp.int32, (MAX_SEG, MAX_SEG), 0)

    for bb in range(B):
        sums_b = sums_ref[bb]
        cnt = jnp.sum(
            hist_ref[bb * WORKERS_PER_B:(bb + 1) * WORKERS_PER_B],
            axis=0, keepdims=True)
        present = (cnt > 0.5).astype(jnp.float32)
        recip = 1.0 / jnp.maximum(cnt, 1.0)

        acc = jax.lax.dot_general(g, sums_b, (((0,), (0,)), ((), ())),
                                  preferred_element_type=hp, precision=hi)
        rows = acc * recip + const

        pos = jax.lax.dot_general(present, tri, (((1,), (0,)), ((), ())),
                                  preferred_element_type=hp, precision=hi)
        pos_i = pos.astype(jnp.int32) - 1
        perm = ((dd == pos_i) & (present > 0.5)).astype(jnp.float32)

        out_ref[bb] = jax.lax.dot_general(
            perm, rows, (((1,), (1,)), ((), ())),
            preferred_element_type=hp, precision=hi)


def kernel(segment_labels, features, seg_table, w1, b1, w2, b2, Wout, bout):
    feats = features.reshape(B, FEAT_DIM, HW)
    labels = segment_labels.reshape(B, 1, HW)
    b1c = b1.reshape(EMBED_DIM, 1)
    b2c = b2.reshape(EMBED_DIM, 1)
    boutc = bout.reshape(EMBED_DIM, 1)

    hist = _sc_histogram(segment_labels.reshape(B * HW))

    grid = (B, HW // PIX_BLK)
    sums = pl.pallas_call(
        _main_body,
        grid=grid,
        in_specs=[
            pl.BlockSpec((1, FEAT_DIM, PIX_BLK), lambda b, t: (b, 0, t)),
            pl.BlockSpec((1, 1, PIX_BLK), lambda b, t: (b, 0, t)),
            pl.BlockSpec((EMBED_DIM, FEAT_DIM), lambda b, t: (0, 0)),
            pl.BlockSpec((EMBED_DIM, 1), lambda b, t: (0, 0)),
        ],
        out_specs=pl.BlockSpec((1, EMBED_DIM, MAX_SEG), lambda b, t: (b, 0, 0)),
        out_shape=jax.ShapeDtypeStruct((B, EMBED_DIM, MAX_SEG), jnp.float32),
        compiler_params=pltpu.CompilerParams(
            dimension_semantics=("arbitrary", "arbitrary")),
    )(feats, labels, w1, b1c)

    out = pl.pallas_call(
        _epilogue_body,
        out_shape=jax.ShapeDtypeStruct((B, MAX_SEG, EMBED_DIM), jnp.float32),
    )(sums, hist, seg_table, w2, b2c, Wout, boutc)
    return out


# speedup vs baseline: 1.0481x; 1.0481x over previous
import functools

import jax
import jax.numpy as jnp
from jax import lax
from jax.experimental import pallas as pl
from jax.experimental.pallas import tpu as pltpu
from jax.experimental.pallas import tpu_sc as plsc

B = 2
H = 512
W = 512
HW = H * W
FEAT_DIM = 96
EMBED_DIM = 64
MAX_SEG = 32

PIX_BLK = 32768

N_WORKERS = 32
CHUNK = B * HW // N_WORKERS
WORKERS_PER_B = N_WORKERS // B


def _sc_hist_body(lab_hbm, out_hbm, lab_v, cnt_v):
    wid = lax.axis_index("s") * 2 + lax.axis_index("c")
    base = wid * CHUNK
    pltpu.sync_copy(lab_hbm.at[pl.ds(base, CHUNK)], lab_v)
    cnt_v[pl.ds(0, 16)] = jnp.zeros((16,), jnp.float32)
    cnt_v[pl.ds(16, 16)] = jnp.zeros((16,), jnp.float32)
    ones = jnp.ones((16,), jnp.float32)

    @pl.loop(0, CHUNK // 16, unroll=8)
    def _count(i):
        lab = lab_v[pl.ds(i * 16, 16)]
        plsc.addupdate_scatter(cnt_v, [lab], ones)

    pltpu.sync_copy(cnt_v, out_hbm.at[wid])


def _sc_histogram(labels_flat):
    mesh = plsc.VectorSubcoreMesh(core_axis_name="c", subcore_axis_name="s")
    k = functools.partial(
        pl.kernel,
        mesh=mesh,
        out_type=jax.ShapeDtypeStruct((N_WORKERS, MAX_SEG), jnp.float32),
        scratch_types=[
            pltpu.VMEM((CHUNK,), jnp.int32),
            pltpu.VMEM((MAX_SEG,), jnp.float32),
        ],
        compiler_params=pltpu.CompilerParams(needs_layout_passes=False),
    )(_sc_hist_body)
    return k(labels_flat)


def _main_body(f_ref, l_ref, w1_ref, b1_ref, sums_ref):
    t = pl.program_id(1)

    @pl.when(t == 0)
    def _init():
        sums_ref[...] = jnp.zeros_like(sums_ref)

    x = f_ref[0].astype(jnp.bfloat16)
    w1 = w1_ref[...].astype(jnp.bfloat16)
    b1 = b1_ref[...]
    h = jax.lax.dot_general(w1, x, (((1,), (0,)), ((), ())),
                            preferred_element_type=jnp.float32)
    h = jnp.maximum(h + b1, 0.0).astype(jnp.bfloat16)

    lab = l_ref[0]
    sid = jax.lax.broadcasted_iota(jnp.int32, (MAX_SEG, PIX_BLK), 0)
    oh = (lab == sid).astype(jnp.bfloat16)

    sums_ref[0] += jax.lax.dot_general(h, oh, (((1,), (1,)), ((), ())),
                                       preferred_element_type=jnp.float32)


def _epilogue_body(sums_ref, hist_ref, table_ref, w2_ref, b2_ref, wout_ref,
                   bout_ref, out_ref):
    w2 = w2_ref[...]
    wout = wout_ref[...]
    wa = wout[:, :EMBED_DIM]
    wb = wout[:, EMBED_DIM:]
    b2 = b2_ref[...]
    bout = bout_ref[...]
    emb = table_ref[...][:MAX_SEG]

    hp = jnp.float32
    hi = jax.lax.Precision.HIGHEST
    g = jax.lax.dot_general(w2, wa, (((0,), (1,)), ((), ())),
                            preferred_element_type=hp, precision=hi)
    const = jax.lax.dot_general(wb, emb, (((1,), (1,)), ((), ())),
                                preferred_element_type=hp, precision=hi)
    const = const + jax.lax.dot_general(
        wa, b2, (((1,), (0,)), ((), ())), preferred_element_type=hp,
        precision=hi) + bout

    jj = jax.lax.broadcasted_iota(jnp.int32, (MAX_SEG, MAX_SEG), 0)
    ii = jax.lax.broadcasted_iota(jnp.int32, (MAX_SEG, MAX_SEG), 1)
    tri = (jj <= ii).astype(jnp.float32)
    dd = jax.lax.broadcasted_iota(jnp.int32, (MAX_SEG, MAX_SEG), 0)

    for bb in range(B):
        sums_b = sums_ref[bb]
        cnt = jnp.sum(
            hist_ref[bb * WORKERS_PER_B:(bb + 1) * WORKERS_PER_B],
            axis=0, keepdims=True)
        present = (cnt > 0.5).astype(jnp.float32)
        recip = 1.0 / jnp.maximum(cnt, 1.0)

        acc = jax.lax.dot_general(g, sums_b, (((0,), (0,)), ((), ())),
                                  preferred_element_type=hp, precision=hi)
        rows = acc * recip + const

        pos = jax.lax.dot_general(present, tri, (((1,), (0,)), ((), ())),
                                  preferred_element_type=hp, precision=hi)
        pos_i = pos.astype(jnp.int32) - 1
        perm = ((dd == pos_i) & (present > 0.5)).astype(jnp.float32)

        out_ref[bb] = jax.lax.dot_general(
            perm, rows, (((1,), (1,)), ((), ())),
            preferred_element_type=hp, precision=hi)


def kernel(segment_labels, features, seg_table, w1, b1, w2, b2, Wout, bout):
    feats = features.reshape(B, FEAT_DIM, HW)
    labels = segment_labels.reshape(B, 1, HW)
    b1c = b1.reshape(EMBED_DIM, 1)
    b2c = b2.reshape(EMBED_DIM, 1)
    boutc = bout.reshape(EMBED_DIM, 1)

    hist = _sc_histogram(segment_labels.reshape(B * HW))

    grid = (B, HW // PIX_BLK)
    sums = pl.pallas_call(
        _main_body,
        grid=grid,
        in_specs=[
            pl.BlockSpec((1, FEAT_DIM, PIX_BLK), lambda b, t: (b, 0, t)),
            pl.BlockSpec((1, 1, PIX_BLK), lambda b, t: (b, 0, t)),
            pl.BlockSpec((EMBED_DIM, FEAT_DIM), lambda b, t: (0, 0)),
            pl.BlockSpec((EMBED_DIM, 1), lambda b, t: (0, 0)),
        ],
        out_specs=pl.BlockSpec((1, EMBED_DIM, MAX_SEG), lambda b, t: (b, 0, 0)),
        out_shape=jax.ShapeDtypeStruct((B, EMBED_DIM, MAX_SEG), jnp.float32),
        compiler_params=pltpu.CompilerParams(
            dimension_semantics=("arbitrary", "arbitrary")),
    )(feats, labels, w1, b1c)

    sums, hist = jax.lax.optimization_barrier((sums, hist))

    out = pl.pallas_call(
        _epilogue_body,
        out_shape=jax.ShapeDtypeStruct((B, MAX_SEG, EMBED_DIM), jnp.float32),
    )(sums, hist, seg_table, w2, b2c, Wout, boutc)
    return out
